# baseline (speedup 1.0000x reference)
import jax
import jax.numpy as jnp
from jax import lax
from jax.experimental import pallas as pl
from jax.experimental.pallas import tpu as pltpu
from jax.experimental.pallas import tpu_sc as plsc

NU, NT = 10000, 100000
NC, NS = 2, 16
NW = NC * NS
BLK = 128

NUP = 10112
NTP = 100096
U_TW = NUP // NS
T_TW = NTP // NS

EFB = 80
EWB = 26
EFP = EFB * NW * BLK
EWP = EWB * NW * BLK

BLKU = 64
CHF, NCHF = 32, 5
CHW, NCHW = 26, 2

BLKW = 64
NBW = EWP // (NS * BLKW)
CHT, NCHT = 26, 4
TP_R = 10112
TP_P = 10
NTPW = TP_R * TP_P
TP_TW = TP_R // NS

_mesh = plsc.VectorSubcoreMesh(
    core_axis_name="c", subcore_axis_name="s", num_cores=NC, num_subcores=NS)


def _lrelu(x):
  return jnp.where(x >= 0, x, 0.01 * x)


def _fill_f32(ref, nrows, ncols, value):
  v = jnp.full((16,), value, jnp.float32)

  @pl.loop(0, nrows)
  def _(i):
    for j in range(ncols // 16):
      ref[i, pl.ds(j * 16, 16)] = v


def _fill_f32_1d(ref, n, value):
  v = jnp.full((16,), value, jnp.float32)

  @pl.loop(0, n // 16)
  def _(i):
    ref[pl.ds(i * 16, 16)] = v


def _sp2hbm_1d(src, dst, off, n, vb):
  for base in range(0, n, 1024):
    m = min(1024, n - base)
    pltpu.sync_copy(src.at[pl.ds(off + base, m)], vb.at[pl.ds(0, m)])
    pltpu.sync_copy(vb.at[pl.ds(0, m)], dst.at[pl.ds(off + base, m)])


def _edge_loop(table, acc, sidx_v, didx_v, nblk, bufs):
  nb_slots = len(bufs)
  for j in range(min(nb_slots, nblk)):
    r, gs, _ = bufs[j]
    pltpu.async_copy(table.at[sidx_v.at[j]], r, gs)

  nouter = (nblk + nb_slots - 1) // nb_slots

  @pl.loop(0, nouter)
  def _(i):
    for j in range(nb_slots):
      r, gs, ss = bufs[j]
      b = i * nb_slots + j

      @pl.when(b < nblk)
      def _():
        pltpu.make_async_copy(table.at[sidx_v.at[b]], r, gs).wait()
        pltpu.async_copy(r, acc.at[didx_v.at[b]], ss, add=True)
        nb = b + nb_slots

        @pl.when(nb < nblk)
        def _():
          pltpu.make_async_copy(r, acc.at[didx_v.at[b]], ss).wait()
          pltpu.async_copy(table.at[sidx_v.at[nb]], r, gs)

  for j in range(nb_slots):
    if nblk > j:
      r, _, ss = bufs[j]
      lj = nblk - 1 - ((nblk - 1 - j) % nb_slots)
      pltpu.make_async_copy(r, acc.at[didx_v.at[lj]], ss).wait()


def _deg_body(sfN, dfN, swN, dwN,
              osf0, odf0, osw0, odw0, osf1, odf1, osw1, odw1,
              isf_v, idf_v, isw_v, idw_v, ones_v, zb_v,
              asf, adf, asw, adw, semA, semB, semC, semD):
  c = lax.axis_index("c")
  s = lax.axis_index("s")
  w = c * NS + s

  _fill_f32_1d(ones_v, BLK, 1.0)
  _fill_f32_1d(zb_v, 1024, 0.0)

  pltpu.sync_copy(zb_v.at[pl.ds(0, U_TW)], asf.at[pl.ds(s * U_TW, U_TW)])
  pltpu.sync_copy(zb_v.at[pl.ds(0, U_TW)], adf.at[pl.ds(s * U_TW, U_TW)])
  pltpu.sync_copy(zb_v.at[pl.ds(0, U_TW)], asw.at[pl.ds(s * U_TW, U_TW)])
  for k in range(6):
    pltpu.sync_copy(zb_v, adw.at[pl.ds(s * T_TW + k * 1024, 1024)])
  pltpu.sync_copy(zb_v.at[pl.ds(0, T_TW - 6144)],
                  adw.at[pl.ds(s * T_TW + 6144, T_TW - 6144)])

  pltpu.sync_copy(sfN.at[w], isf_v)
  pltpu.sync_copy(dfN.at[w], idf_v)
  pltpu.sync_copy(swN.at[w], isw_v)
  pltpu.sync_copy(dwN.at[w], idw_v)

  plsc.subcore_barrier()

  @pl.loop(0, EFB)
  def _(b):
    pltpu.async_copy(ones_v, asf.at[isf_v.at[b]], semA, add=True)
    pltpu.async_copy(ones_v, adf.at[idf_v.at[b]], semB, add=True)

  @pl.loop(0, EWB)
  def _(b):
    pltpu.async_copy(ones_v, asw.at[isw_v.at[b]], semC, add=True)
    pltpu.async_copy(ones_v, adw.at[idw_v.at[b]], semD, add=True)

  @pl.loop(0, EFB)
  def _(b):
    pltpu.make_async_copy(ones_v, asf.at[isf_v.at[b]], semA).wait()
    pltpu.make_async_copy(ones_v, adf.at[idf_v.at[b]], semB).wait()

  @pl.loop(0, EWB)
  def _(b):
    pltpu.make_async_copy(ones_v, asw.at[isw_v.at[b]], semC).wait()
    pltpu.make_async_copy(ones_v, adw.at[idw_v.at[b]], semD).wait()

  plsc.subcore_barrier()

  @pl.when(c == 0)
  def _():
    _sp2hbm_1d(asf, osf0, s * U_TW, U_TW, zb_v)
    _sp2hbm_1d(adf, odf0, s * U_TW, U_TW, zb_v)
    _sp2hbm_1d(asw, osw0, s * U_TW, U_TW, zb_v)
    _sp2hbm_1d(adw, odw0, s * T_TW, T_TW, zb_v)

  @pl.when(c == 1)
  def _():
    _sp2hbm_1d(asf, osf1, s * U_TW, U_TW, zb_v)
    _sp2hbm_1d(adf, odf1, s * U_TW, U_TW, zb_v)
    _sp2hbm_1d(asw, osw1, s * U_TW, U_TW, zb_v)
    _sp2hbm_1d(adw, odw1, s * T_TW, T_TW, zb_v)


_deg_kernel = pl.kernel(
    _deg_body,
    out_type=[jax.ShapeDtypeStruct((NUP,), jnp.float32),
              jax.ShapeDtypeStruct((NUP,), jnp.float32),
              jax.ShapeDtypeStruct((NUP,), jnp.float32),
              jax.ShapeDtypeStruct((NTP,), jnp.float32),
              jax.ShapeDtypeStruct((NUP,), jnp.float32),
              jax.ShapeDtypeStruct((NUP,), jnp.float32),
              jax.ShapeDtypeStruct((NUP,), jnp.float32),
              jax.ShapeDtypeStruct((NTP,), jnp.float32)],
    mesh=_mesh,
    scratch_types=[pltpu.VMEM((EFB, BLK), jnp.int32),
                   pltpu.VMEM((EFB, BLK), jnp.int32),
                   pltpu.VMEM((EWB, BLK), jnp.int32),
                   pltpu.VMEM((EWB, BLK), jnp.int32),
                   pltpu.VMEM((BLK,), jnp.float32),
                   pltpu.VMEM((1024,), jnp.float32),
                   pltpu.VMEM_SHARED((NUP,), jnp.float32),
                   pltpu.VMEM_SHARED((NUP,), jnp.float32),
                   pltpu.VMEM_SHARED((NUP,), jnp.float32),
                   pltpu.VMEM_SHARED((NTP,), jnp.float32),
                   pltpu.SemaphoreType.DMA,
                   pltpu.SemaphoreType.DMA,
                   pltpu.SemaphoreType.DMA,
                   pltpu.SemaphoreType.DMA],
    name="sc_degrees")


def _make_agg_user(nch, ch, name):
  nfull = U_TW // BLKU
  rem = U_TW - nfull * BLKU

  def body(table, sidx, didx, out, sidx_v, didx_v, r0, r1, r2, r3, acc,
           gs0, gs1, gs2, gs3, ss0, ss1, ss2, ss3):
    c = lax.axis_index("c")
    s = lax.axis_index("s")
    w = c * NS + s

    _fill_f32(r0, BLKU, 128, 0.0)
    for k in range(nfull):
      pltpu.sync_copy(r0, acc.at[pl.ds(s * U_TW + k * BLKU, BLKU)])
    pltpu.sync_copy(r0.at[pl.ds(0, rem)],
                    acc.at[pl.ds(s * U_TW + nfull * BLKU, rem)])
    plsc.subcore_barrier()

    bufs = ((r0, gs0, ss0), (r1, gs1, ss1), (r2, gs2, ss2), (r3, gs3, ss3))

    @pl.loop(0, nch)
    def _(ci):
      pltpu.sync_copy(sidx.at[w].at[ci], sidx_v)
      pltpu.sync_copy(didx.at[w].at[ci], didx_v)
      _edge_loop(table, acc, sidx_v, didx_v, ch, bufs)

    plsc.subcore_barrier()
    for k in range(nfull):
      pltpu.sync_copy(acc.at[pl.ds(s * U_TW + k * BLKU, BLKU)], r0)
      pltpu.sync_copy(r0, out.at[c].at[pl.ds(s * U_TW + k * BLKU, BLKU)])
    pltpu.sync_copy(acc.at[pl.ds(s * U_TW + nfull * BLKU, rem)],
                    r0.at[pl.ds(0, rem)])
    pltpu.sync_copy(r0.at[pl.ds(0, rem)],
                    out.at[c].at[pl.ds(s * U_TW + nfull * BLKU, rem)])

  return pl.kernel(
      body,
      out_type=jax.ShapeDtypeStruct((NC, NUP, 128), jnp.float32),
      mesh=_mesh,
      scratch_types=[pltpu.VMEM((ch, BLKU), jnp.int32),
                     pltpu.VMEM((ch, BLKU), jnp.int32),
                     pltpu.VMEM((BLKU, 128), jnp.float32),
                     pltpu.VMEM((BLKU, 128), jnp.float32),
                     pltpu.VMEM((BLKU, 128), jnp.float32),
                     pltpu.VMEM((BLKU, 128), jnp.float32),
                     pltpu.VMEM_SHARED((NUP, 128), jnp.float32),
                     pltpu.SemaphoreType.DMA,
                     pltpu.SemaphoreType.DMA,
                     pltpu.SemaphoreType.DMA,
                     pltpu.SemaphoreType.DMA,
                     pltpu.SemaphoreType.DMA,
                     pltpu.SemaphoreType.DMA,
                     pltpu.SemaphoreType.DMA,
                     pltpu.SemaphoreType.DMA],
      name=name)


_agg_follows = _make_agg_user(NCHF, CHF, "sc_agg_follows")
_agg_rev = _make_agg_user(NCHW, CHW, "sc_agg_rev_writes")


def _aggw_body(table, sidx, didx, out, sidx_v, didx_v, dloc_v,
               r0, r1, r2, r3, acc,
               gs0, gs1, gs2, gs3, ss0, ss1, ss2, ss3):
  c = lax.axis_index("c")
  s = lax.axis_index("s")
  nfull = TP_TW // BLKW
  rem = TP_TW - nfull * BLKW
  bufs = ((r0, gs0, ss0), (r1, gs1, ss1), (r2, gs2, ss2), (r3, gs3, ss3))

  for t in range(TP_P // NC):
    p = t * NC + c
    lo = p * TP_R

    _fill_f32(r0, BLKW, 128, 0.0)

    @pl.loop(0, nfull)
    def _(m):
      pltpu.sync_copy(r0, acc.at[pl.ds(s * TP_TW + m * BLKW, BLKW)])

    pltpu.sync_copy(r0.at[pl.ds(0, rem)],
                    acc.at[pl.ds(s * TP_TW + nfull * BLKW, rem)])
    plsc.subcore_barrier()

    @pl.loop(0, NCHT)
    def _(ci):
      pltpu.sync_copy(sidx.at[s].at[ci], sidx_v)
      pltpu.sync_copy(didx.at[s].at[ci], didx_v)

      @pl.loop(0, CHT)
      def _(bi):
        for j in range(BLKW // 16):
          d = didx_v[bi, pl.ds(j * 16, 16)]
          inr = (d >= lo) & (d < lo + TP_R)
          dloc_v[bi, pl.ds(j * 16, 16)] = jnp.where(inr, d - lo, TP_R)

      _edge_loop(table, acc, sidx_v, dloc_v, CHT, bufs)

    plsc.subcore_barrier()

    @pl.loop(0, nfull)
    def _(m):
      pltpu.sync_copy(acc.at[pl.ds(s * TP_TW + m * BLKW, BLKW)], r0)
      pltpu.sync_copy(r0, out.at[pl.ds(lo + s * TP_TW + m * BLKW, BLKW)])

    pltpu.sync_copy(acc.at[pl.ds(s * TP_TW + nfull * BLKW, rem)],
                    r0.at[pl.ds(0, rem)])
    pltpu.sync_copy(r0.at[pl.ds(0, rem)],
                    out.at[pl.ds(lo + s * TP_TW + nfull * BLKW, rem)])


_agg_writes = pl.kernel(
    _aggw_body,
    out_type=jax.ShapeDtypeStruct((NTPW, 128), jnp.float32),
    mesh=_mesh,
    scratch_types=[pltpu.VMEM((CHT, BLKW), jnp.int32),
                   pltpu.VMEM((CHT, BLKW), jnp.int32),
                   pltpu.VMEM((CHT, BLKW), jnp.int32),
                   pltpu.VMEM((BLKW, 128), jnp.float32),
                   pltpu.VMEM((BLKW, 128), jnp.float32),
                   pltpu.VMEM((BLKW, 128), jnp.float32),
                   pltpu.VMEM((BLKW, 128), jnp.float32),
                   pltpu.VMEM_SHARED((TP_R + 8, 128), jnp.float32),
                   pltpu.SemaphoreType.DMA,
                   pltpu.SemaphoreType.DMA,
                   pltpu.SemaphoreType.DMA,
                   pltpu.SemaphoreType.DMA,
                   pltpu.SemaphoreType.DMA,
                   pltpu.SemaphoreType.DMA,
                   pltpu.SemaphoreType.DMA,
                   pltpu.SemaphoreType.DMA],
    name="sc_agg_writes")


BU = 1000
BT = 1000


def _norm_body(sf0, sf1, df0, df1, sw0, sw1, dw0, dw1, osf, odf, osw, odw):
  for a, b, o_ref in ((sf0, sf1, osf), (df0, df1, odf),
                      (sw0, sw1, osw), (dw0, dw1, odw)):
    d = a[...] + b[...]
    o_ref[...] = lax.rsqrt(jnp.maximum(d, 1.0))


_norm_kernel = pl.pallas_call(
    _norm_body,
    out_shape=[jax.ShapeDtypeStruct((1, NUP), jnp.float32),
               jax.ShapeDtypeStruct((1, NUP), jnp.float32),
               jax.ShapeDtypeStruct((1, NUP), jnp.float32),
               jax.ShapeDtypeStruct((1, NTP), jnp.float32)])


def _user_dense_body(ud, un, uc, wd, bd, wn, bn, wc, bc, w10, w11, isf, isw,
                     mu0_o, mu1_o):
  d = _lrelu(jnp.dot(ud[...], wd[...], preferred_element_type=jnp.float32)
             + bd[...])
  n = _lrelu(jnp.dot(un[...], wn[...], preferred_element_type=jnp.float32)
             + bn[...])
  cm = _lrelu(jnp.dot(uc[...], wc[...], preferred_element_type=jnp.float32)
              + bc[...])
  x = jnp.concatenate([d, n, cm], axis=1)
  mu0_o[...] = jnp.dot(x, w10[...], preferred_element_type=jnp.float32) \
      * isf[...]
  mu1_o[...] = jnp.dot(x, w11[...], preferred_element_type=jnp.float32) \
      * isw[...]


_user_dense = pl.pallas_call(
    _user_dense_body,
    grid=(NU // BU,),
    in_specs=[
        pl.BlockSpec((BU, 100), lambda i: (i, 0)),
        pl.BlockSpec((BU, 6), lambda i: (i, 0)),
        pl.BlockSpec((BU, 11), lambda i: (i, 0)),
        pl.BlockSpec((100, 64), lambda i: (0, 0)),
        pl.BlockSpec((1, 64), lambda i: (0, 0)),
        pl.BlockSpec((6, 32), lambda i: (0, 0)),
        pl.BlockSpec((1, 32), lambda i: (0, 0)),
        pl.BlockSpec((11, 32), lambda i: (0, 0)),
        pl.BlockSpec((1, 32), lambda i: (0, 0)),
        pl.BlockSpec((128, 128), lambda i: (0, 0)),
        pl.BlockSpec((128, 128), lambda i: (0, 0)),
        pl.BlockSpec((BU, 1), lambda i: (i, 0)),
        pl.BlockSpec((BU, 1), lambda i: (i, 0)),
    ],
    out_specs=[pl.BlockSpec((BU, 128), lambda i: (i, 0)),
               pl.BlockSpec((BU, 128), lambda i: (i, 0))],
    out_shape=[jax.ShapeDtypeStruct((NU, 128), jnp.float32),
               jax.ShapeDtypeStruct((NU, 128), jnp.float32)])


def _tweet_dense_body(tx, wt, bt_, w12, idw, mt2_o):
  xt = _lrelu(jnp.dot(tx[...], wt[...], preferred_element_type=jnp.float32)
              + bt_[...])
  mt2_o[...] = jnp.dot(xt, w12[...], preferred_element_type=jnp.float32) \
      * idw[...]


_tweet_dense = pl.pallas_call(
    _tweet_dense_body,
    grid=(NT // BT,),
    in_specs=[
        pl.BlockSpec((BT, 100), lambda i: (i, 0)),
        pl.BlockSpec((100, 128), lambda i: (0, 0)),
        pl.BlockSpec((1, 128), lambda i: (0, 0)),
        pl.BlockSpec((128, 128), lambda i: (0, 0)),
        pl.BlockSpec((BT, 1), lambda i: (i, 0)),
    ],
    out_specs=pl.BlockSpec((BT, 128), lambda i: (i, 0)),
    out_shape=jax.ShapeDtypeStruct((NT, 128), jnp.float32))


def _user_mid_body(aggF, aggR, idf, isw, b10, b12, w20, w21, isf,
                   mu0_o, mu1_o):
  u1 = (aggF[0] + aggF[1]) * idf[...] + (aggR[0] + aggR[1]) * isw[...] \
      + b10[...] + b12[...]
  mu0_o[...] = jnp.dot(u1, w20[...], preferred_element_type=jnp.float32) \
      * isf[...]
  mu1_o[...] = jnp.dot(u1, w21[...], preferred_element_type=jnp.float32) \
      * isw[...]


_user_mid = pl.pallas_call(
    _user_mid_body,
    grid=(NU // BU,),
    in_specs=[
        pl.BlockSpec((NC, BU, 128), lambda i: (0, i, 0)),
        pl.BlockSpec((NC, BU, 128), lambda i: (0, i, 0)),
        pl.BlockSpec((BU, 1), lambda i: (i, 0)),
        pl.BlockSpec((BU, 1), lambda i: (i, 0)),
        pl.BlockSpec((1, 128), lambda i: (0, 0)),
        pl.BlockSpec((1, 128), lambda i: (0, 0)),
        pl.BlockSpec((128, 128), lambda i: (0, 0)),
        pl.BlockSpec((128, 128), lambda i: (0, 0)),
        pl.BlockSpec((BU, 1), lambda i: (i, 0)),
    ],
    out_specs=[pl.BlockSpec((BU, 128), lambda i: (i, 0)),
               pl.BlockSpec((BU, 128), lambda i: (i, 0))],
    out_shape=[jax.ShapeDtypeStruct((NU, 128), jnp.float32),
               jax.ShapeDtypeStruct((NU, 128), jnp.float32)])


def _tweet_mid_body(aggW, idw, b11, w22, mt2_o):
  t1 = aggW[...] * idw[...] + b11[...]
  mt2_o[...] = jnp.dot(t1, w22[...], preferred_element_type=jnp.float32) \
      * idw[...]


_tweet_mid = pl.pallas_call(
    _tweet_mid_body,
    grid=(NT // BT,),
    in_specs=[
        pl.BlockSpec((BT, 128), lambda i: (i, 0)),
        pl.BlockSpec((BT, 1), lambda i: (i, 0)),
        pl.BlockSpec((1, 128), lambda i: (0, 0)),
        pl.BlockSpec((128, 128), lambda i: (0, 0)),
    ],
    out_specs=pl.BlockSpec((BT, 128), lambda i: (i, 0)),
    out_shape=jax.ShapeDtypeStruct((NT, 128), jnp.float32))


def _user_head_body(aggF, aggR, idf, isw, b20, b22, wo10, bo10, wo20, bo20,
                    out_o):
  u2 = (aggF[0] + aggF[1]) * idf[...] + (aggR[0] + aggR[1]) * isw[...] \
      + b20[...] + b22[...]
  o = _lrelu(jnp.dot(u2, wo10[...], preferred_element_type=jnp.float32)
             + bo10[...])
  out_o[...] = jnp.dot(o, wo20[...], preferred_element_type=jnp.float32) \
      + bo20[...]


_user_head = pl.pallas_call(
    _user_head_body,
    grid=(NU // BU,),
    in_specs=[
        pl.BlockSpec((NC, BU, 128), lambda i: (0, i, 0)),
        pl.BlockSpec((NC, BU, 128), lambda i: (0, i, 0)),
        pl.BlockSpec((BU, 1), lambda i: (i, 0)),
        pl.BlockSpec((BU, 1), lambda i: (i, 0)),
        pl.BlockSpec((1, 128), lambda i: (0, 0)),
        pl.BlockSpec((1, 128), lambda i: (0, 0)),
        pl.BlockSpec((128, 128), lambda i: (0, 0)),
        pl.BlockSpec((1, 128), lambda i: (0, 0)),
        pl.BlockSpec((128, 2), lambda i: (0, 0)),
        pl.BlockSpec((1, 2), lambda i: (0, 0)),
    ],
    out_specs=pl.BlockSpec((BU, 2), lambda i: (i, 0)),
    out_shape=jax.ShapeDtypeStruct((NU, 2), jnp.float32))


def _tweet_head_body(aggW, idw, b21, wo11, bo11, wo21, bo21, out_o):
  t2 = aggW[...] * idw[...] + b21[...]
  o = _lrelu(jnp.dot(t2, wo11[...], preferred_element_type=jnp.float32)
             + bo11[...])
  out_o[...] = jnp.dot(o, wo21[...], preferred_element_type=jnp.float32) \
      + bo21[...]


_tweet_head = pl.pallas_call(
    _tweet_head_body,
    grid=(NT // BT,),
    in_specs=[
        pl.BlockSpec((BT, 128), lambda i: (i, 0)),
        pl.BlockSpec((BT, 1), lambda i: (i, 0)),
        pl.BlockSpec((1, 128), lambda i: (0, 0)),
        pl.BlockSpec((128, 128), lambda i: (0, 0)),
        pl.BlockSpec((1, 128), lambda i: (0, 0)),
        pl.BlockSpec((128, 2), lambda i: (0, 0)),
        pl.BlockSpec((1, 2), lambda i: (0, 0)),
    ],
    out_specs=pl.BlockSpec((BT, 2), lambda i: (i, 0)),
    out_shape=jax.ShapeDtypeStruct((NT, 2), jnp.float32))


def kernel(user_des, user_num, user_cat, tweet_x, Wd, bd, Wn, bn, Wc, bc,
           Wt, bt, W1, b1, W2, b2, Wo1, bo1, Wo2, bo2,
           edge_index_follows, edge_src_writes, edge_dst_writes):
  sf = edge_index_follows[0]
  df = edge_index_follows[1]
  sw = edge_src_writes
  dw = edge_dst_writes

  def padf(a, total, val):
    p = jnp.full((total - a.shape[0],), val, jnp.int32)
    return jnp.concatenate([a, p])

  sf_pad0 = padf(sf, EFP, 0)
  df_padN = padf(df, EFP, NU)
  sw_pad0 = padf(sw, EWP, 0)
  sw_padN = padf(sw, EWP, NU)
  dw_pad0 = padf(dw, EWP, 0)
  dw_padT = padf(dw, EWP, NT)

  sfN_d = padf(sf, EFP, NU).reshape(NW, EFB, BLK)
  dfN_d = df_padN.reshape(NW, EFB, BLK)
  swN_d = sw_padN.reshape(NW, EWB, BLK)
  dwN_d = dw_padT.reshape(NW, EWB, BLK)
  sf0_a = sf_pad0.reshape(NW, NCHF, CHF, BLKU)
  dfN_a = df_padN.reshape(NW, NCHF, CHF, BLKU)
  dw0_a = dw_pad0.reshape(NW, NCHW, CHW, BLKU)
  swN_a = sw_padN.reshape(NW, NCHW, CHW, BLKU)
  sw0_w = sw_pad0.reshape(NS, NCHT, CHT, BLKW)
  dwT_w = dw_padT.reshape(NS, NCHT, CHT, BLKW)

  dsf0, ddf0, dsw0, ddw0, dsf1, ddf1, dsw1, ddw1 = _deg_kernel(
      sfN_d, dfN_d, swN_d, dwN_d)
  r = lambda v: v.reshape(1, -1)
  nsf, ndf, nsw, ndw = _norm_kernel(r(dsf0), r(dsf1), r(ddf0), r(ddf1),
                                    r(dsw0), r(dsw1), r(ddw0), r(ddw1))
  isf = nsf.reshape(NUP, 1)
  idf = ndf.reshape(NUP, 1)
  isw = nsw.reshape(NUP, 1)
  idw = ndw.reshape(NTP, 1)

  mu0, mu1 = _user_dense(user_des, user_num, user_cat, Wd, r(bd), Wn, r(bn),
                         Wc, r(bc), W1[0], W1[1], isf, isw)
  mt2 = _tweet_dense(tweet_x, Wt, r(bt), W1[2], idw)

  aggF1 = _agg_follows(mu0, sf0_a, dfN_a)
  aggR1 = _agg_rev(mt2, dw0_a, swN_a)
  aggW1 = _agg_writes(mu1, sw0_w, dwT_w)

  mu0_2, mu1_2 = _user_mid(aggF1, aggR1, idf, isw, r(b1[0]), r(b1[2]),
                           W2[0], W2[1], isf)
  mt2_2 = _tweet_mid(aggW1, idw, r(b1[1]), W2[2])

  aggF2 = _agg_follows(mu0_2, sf0_a, dfN_a)
  aggR2 = _agg_rev(mt2_2, dw0_a, swN_a)
  aggW2 = _agg_writes(mu1_2, sw0_w, dwT_w)

  out_u = _user_head(aggF2, aggR2, idf, isw, r(b2[0]), r(b2[2]),
                     Wo1[0], r(bo1[0]), Wo2[0], r(bo2[0]))
  out_t = _tweet_head(aggW2, idw, r(b2[1]), Wo1[1], r(bo1[1]),
                      Wo2[1], r(bo2[1]))

  return jnp.concatenate([out_u, out_t], axis=0)


# speedup vs baseline: 2.4649x; 2.4649x over previous
import jax
import jax.numpy as jnp
from jax import lax
from jax.experimental import pallas as pl
from jax.experimental.pallas import tpu as pltpu
from jax.experimental.pallas import tpu_sc as plsc

NU, NT = 10000, 100000
NC, NS = 2, 16
NW = NC * NS
BLK = 128

NUP = 10112
NTP = 100096
U_TW = NUP // NS
T_TW = NTP // NS

EFB = 80
EWB = 26
EFP = EFB * NW * BLK
EWP = EWB * NW * BLK

BLKU = 128
CHF, NCHF = 16, 5
CHW, NCHW = 13, 2

BLKW = 64
NBWS = EWP // NS
NBK = NBWS // BLKW
CAP = 6784
TP_R = 10112
TP_P = 10
NTPW = TP_R * TP_P
TP_TW = TP_R // NS

_mesh = plsc.VectorSubcoreMesh(
    core_axis_name="c", subcore_axis_name="s", num_cores=NC, num_subcores=NS)


def _lrelu(x):
  return jnp.where(x >= 0, x, 0.01 * x)


def _fill_f32(ref, nrows, ncols, value):
  v = jnp.full((16,), value, jnp.float32)

  @pl.loop(0, nrows)
  def _(i):
    for j in range(ncols // 16):
      ref[i, pl.ds(j * 16, 16)] = v


def _fill_f32_1d(ref, n, value):
  v = jnp.full((16,), value, jnp.float32)

  @pl.loop(0, n // 16)
  def _(i):
    ref[pl.ds(i * 16, 16)] = v


def _sp2hbm_1d(src, dst, off, n, vb):
  for base in range(0, n, 1024):
    m = min(1024, n - base)
    pltpu.sync_copy(src.at[pl.ds(off + base, m)], vb.at[pl.ds(0, m)])
    pltpu.sync_copy(vb.at[pl.ds(0, m)], dst.at[pl.ds(off + base, m)])


def _edge_loop(table, acc, sidx_v, didx_v, nblk, bufs):
  nb_slots = len(bufs)
  for j in range(min(nb_slots, nblk)):
    r, gs, _ = bufs[j]
    pltpu.async_copy(table.at[sidx_v.at[j]], r, gs)

  nouter = (nblk + nb_slots - 1) // nb_slots

  @pl.loop(0, nouter)
  def _(i):
    for j in range(nb_slots):
      r, gs, ss = bufs[j]
      b = i * nb_slots + j

      @pl.when(b < nblk)
      def _():
        pltpu.make_async_copy(table.at[sidx_v.at[b]], r, gs).wait()
        pltpu.async_copy(r, acc.at[didx_v.at[b]], ss, add=True)
        nb = b + nb_slots

        @pl.when(nb < nblk)
        def _():
          pltpu.make_async_copy(r, acc.at[didx_v.at[b]], ss).wait()
          pltpu.async_copy(table.at[sidx_v.at[nb]], r, gs)

  for j in range(nb_slots):
    if nblk > j:
      r, _, ss = bufs[j]
      lj = nblk - 1 - ((nblk - 1 - j) % nb_slots)
      pltpu.make_async_copy(r, acc.at[didx_v.at[lj]], ss).wait()


def _deg_body(sfN, dfN, swN, dwN,
              osf0, odf0, osw0, odw0, osf1, odf1, osw1, odw1,
              isf_v, idf_v, isw_v, idw_v, ones_v, zb_v,
              asf, adf, asw, adw, semA, semB, semC, semD):
  c = lax.axis_index("c")
  s = lax.axis_index("s")
  w = c * NS + s

  _fill_f32_1d(ones_v, BLK, 1.0)
  _fill_f32_1d(zb_v, 1024, 0.0)

  pltpu.sync_copy(zb_v.at[pl.ds(0, U_TW)], asf.at[pl.ds(s * U_TW, U_TW)])
  pltpu.sync_copy(zb_v.at[pl.ds(0, U_TW)], adf.at[pl.ds(s * U_TW, U_TW)])
  pltpu.sync_copy(zb_v.at[pl.ds(0, U_TW)], asw.at[pl.ds(s * U_TW, U_TW)])
  for k in range(6):
    pltpu.sync_copy(zb_v, adw.at[pl.ds(s * T_TW + k * 1024, 1024)])
  pltpu.sync_copy(zb_v.at[pl.ds(0, T_TW - 6144)],
                  adw.at[pl.ds(s * T_TW + 6144, T_TW - 6144)])

  pltpu.sync_copy(sfN.at[w], isf_v)
  pltpu.sync_copy(dfN.at[w], idf_v)
  pltpu.sync_copy(swN.at[w], isw_v)
  pltpu.sync_copy(dwN.at[w], idw_v)

  plsc.subcore_barrier()

  @pl.loop(0, EFB)
  def _(b):
    pltpu.async_copy(ones_v, asf.at[isf_v.at[b]], semA, add=True)
    pltpu.async_copy(ones_v, adf.at[idf_v.at[b]], semB, add=True)

  @pl.loop(0, EWB)
  def _(b):
    pltpu.async_copy(ones_v, asw.at[isw_v.at[b]], semC, add=True)
    pltpu.async_copy(ones_v, adw.at[idw_v.at[b]], semD, add=True)

  @pl.loop(0, EFB)
  def _(b):
    pltpu.make_async_copy(ones_v, asf.at[isf_v.at[b]], semA).wait()
    pltpu.make_async_copy(ones_v, adf.at[idf_v.at[b]], semB).wait()

  @pl.loop(0, EWB)
  def _(b):
    pltpu.make_async_copy(ones_v, asw.at[isw_v.at[b]], semC).wait()
    pltpu.make_async_copy(ones_v, adw.at[idw_v.at[b]], semD).wait()

  plsc.subcore_barrier()

  @pl.when(c == 0)
  def _():
    _sp2hbm_1d(asf, osf0, s * U_TW, U_TW, zb_v)
    _sp2hbm_1d(adf, odf0, s * U_TW, U_TW, zb_v)
    _sp2hbm_1d(asw, osw0, s * U_TW, U_TW, zb_v)
    _sp2hbm_1d(adw, odw0, s * T_TW, T_TW, zb_v)

  @pl.when(c == 1)
  def _():
    _sp2hbm_1d(asf, osf1, s * U_TW, U_TW, zb_v)
    _sp2hbm_1d(adf, odf1, s * U_TW, U_TW, zb_v)
    _sp2hbm_1d(asw, osw1, s * U_TW, U_TW, zb_v)
    _sp2hbm_1d(adw, odw1, s * T_TW, T_TW, zb_v)


_deg_kernel = pl.kernel(
    _deg_body,
    out_type=[jax.ShapeDtypeStruct((NUP,), jnp.float32),
              jax.ShapeDtypeStruct((NUP,), jnp.float32),
              jax.ShapeDtypeStruct((NUP,), jnp.float32),
              jax.ShapeDtypeStruct((NTP,), jnp.float32),
              jax.ShapeDtypeStruct((NUP,), jnp.float32),
              jax.ShapeDtypeStruct((NUP,), jnp.float32),
              jax.ShapeDtypeStruct((NUP,), jnp.float32),
              jax.ShapeDtypeStruct((NTP,), jnp.float32)],
    mesh=_mesh,
    scratch_types=[pltpu.VMEM((EFB, BLK), jnp.int32),
                   pltpu.VMEM((EFB, BLK), jnp.int32),
                   pltpu.VMEM((EWB, BLK), jnp.int32),
                   pltpu.VMEM((EWB, BLK), jnp.int32),
                   pltpu.VMEM((BLK,), jnp.float32),
                   pltpu.VMEM((1024,), jnp.float32),
                   pltpu.VMEM_SHARED((NUP,), jnp.float32),
                   pltpu.VMEM_SHARED((NUP,), jnp.float32),
                   pltpu.VMEM_SHARED((NUP,), jnp.float32),
                   pltpu.VMEM_SHARED((NTP,), jnp.float32),
                   pltpu.SemaphoreType.DMA,
                   pltpu.SemaphoreType.DMA,
                   pltpu.SemaphoreType.DMA,
                   pltpu.SemaphoreType.DMA],
    name="sc_degrees")


def _make_agg_user(nch, ch, name):
  nfull = U_TW // BLKU
  rem = U_TW - nfull * BLKU

  def body(table, sidx, didx, out, sidx_v, didx_v, r0, r1, acc,
           gs0, gs1, ss0, ss1):
    c = lax.axis_index("c")
    s = lax.axis_index("s")
    w = c * NS + s

    _fill_f32(r0, BLKU, 128, 0.0)
    for k in range(nfull):
      pltpu.sync_copy(r0, acc.at[pl.ds(s * U_TW + k * BLKU, BLKU)])
    pltpu.sync_copy(r0.at[pl.ds(0, rem)],
                    acc.at[pl.ds(s * U_TW + nfull * BLKU, rem)])
    plsc.subcore_barrier()

    bufs = ((r0, gs0, ss0), (r1, gs1, ss1))

    @pl.loop(0, nch)
    def _(ci):
      pltpu.sync_copy(sidx.at[w].at[ci], sidx_v)
      pltpu.sync_copy(didx.at[w].at[ci], didx_v)
      _edge_loop(table, acc, sidx_v, didx_v, ch, bufs)

    plsc.subcore_barrier()
    for k in range(nfull):
      pltpu.sync_copy(acc.at[pl.ds(s * U_TW + k * BLKU, BLKU)], r0)
      pltpu.sync_copy(r0, out.at[c].at[pl.ds(s * U_TW + k * BLKU, BLKU)])
    pltpu.sync_copy(acc.at[pl.ds(s * U_TW + nfull * BLKU, rem)],
                    r0.at[pl.ds(0, rem)])
    pltpu.sync_copy(r0.at[pl.ds(0, rem)],
                    out.at[c].at[pl.ds(s * U_TW + nfull * BLKU, rem)])

  return pl.kernel(
      body,
      out_type=jax.ShapeDtypeStruct((NC, NUP, 128), jnp.float32),
      mesh=_mesh,
      scratch_types=[pltpu.VMEM((ch, BLKU), jnp.int32),
                     pltpu.VMEM((ch, BLKU), jnp.int32),
                     pltpu.VMEM((BLKU, 128), jnp.float32),
                     pltpu.VMEM((BLKU, 128), jnp.float32),
                     pltpu.VMEM_SHARED((NUP, 128), jnp.float32),
                     pltpu.SemaphoreType.DMA,
                     pltpu.SemaphoreType.DMA,
                     pltpu.SemaphoreType.DMA,
                     pltpu.SemaphoreType.DMA],
      name=name)


_agg_follows = _make_agg_user(NCHF, CHF, "sc_agg_follows")
_agg_rev = _make_agg_user(NCHW, CHW, "sc_agg_rev_writes")


def _fill_i32_1d(ref, n, value):
  v = jnp.full((16,), value, jnp.int32)

  @pl.loop(0, n // 16)
  def _(i):
    ref[pl.ds(i * 16, 16)] = v


def _bin_body(sidx, didx, bsrc, bdst, bcnt,
              sidx_v, didx_v, csrc_v, cdst_v, cd2_v, cnt0_v, cnt1_v):
  c = lax.axis_index("c")
  s = lax.axis_index("s")

  @pl.when(c == 0)
  def _():
    pltpu.sync_copy(sidx.at[s], sidx_v)
    pltpu.sync_copy(didx.at[s], didx_v)
    _fill_i32_1d(csrc_v, CAP, 0)
    _fill_i32_1d(cnt0_v.at[0], 16, 0)
    _fill_i32_1d(cnt1_v.at[0], 16, 0)
    lane = lax.iota(jnp.int32, 16)
    one = jnp.full((16,), 1, jnp.int32)
    zero = jnp.full((16,), 0, jnp.int32)

    for p in range(TP_P):
      lo = p * TP_R
      _fill_i32_1d(cdst_v, CAP, TP_R)

      def step(i, off_v):
        r = i // 8
        col = (i % 8) * 16
        sv = sidx_v[r, pl.ds(col, 16)]
        dv = didx_v[r, pl.ds(col, 16)]
        m = (dv >= lo) & (dv < lo + TP_R)
        cs = plsc.cumsum(jnp.where(m, one, zero))
        idx = jnp.where(m, off_v + cs - one, (CAP - 16) + lane)
        plsc.store_scatter(csrc_v, [idx], sv)
        plsc.store_scatter(cdst_v, [idx], dv - lo)
        return off_v + plsc.all_reduce_population_count(m)

      off_v = lax.fori_loop(0, NBWS // 16, step,
                            jnp.full((16,), 0, jnp.int32))
      nbv = (off_v + (BLKW - 1)) // BLKW
      if p < TP_P // NC:
        cnt0_v[p, pl.ds(0, 16)] = nbv
      else:
        cnt1_v[p - TP_P // NC, pl.ds(0, 16)] = nbv

      @pl.loop(0, NBK)
      def _(b):
        for j in range(BLKW // 16):
          cd2_v[b, pl.ds(j * 16, 16)] = cdst_v[pl.ds(b * BLKW + j * 16, 16)]

      pltpu.sync_copy(csrc_v.at[pl.ds(0, NBWS)], bsrc.at[s].at[p])
      pltpu.sync_copy(cd2_v, bdst.at[s].at[p])

    pltpu.sync_copy(cnt0_v, bcnt.at[0].at[s])
    pltpu.sync_copy(cnt1_v, bcnt.at[1].at[s])


_bin_writes = pl.kernel(
    _bin_body,
    out_type=[jax.ShapeDtypeStruct((NS, TP_P, NBWS), jnp.int32),
              jax.ShapeDtypeStruct((NS, TP_P, NBK, BLKW), jnp.int32),
              jax.ShapeDtypeStruct((NC, NS, 8, 16), jnp.int32)],
    mesh=_mesh,
    scratch_types=[pltpu.VMEM((NBWS // 128, 128), jnp.int32),
                   pltpu.VMEM((NBWS // 128, 128), jnp.int32),
                   pltpu.VMEM((CAP,), jnp.int32),
                   pltpu.VMEM((CAP,), jnp.int32),
                   pltpu.VMEM((NBK, BLKW), jnp.int32),
                   pltpu.VMEM((8, 16), jnp.int32),
                   pltpu.VMEM((8, 16), jnp.int32)],
    compiler_params=pltpu.CompilerParams(needs_layout_passes=False),
    name="sc_bin_writes")


def _edge_loop_dyn(table, acc, srow, drow, nb, maxb, bufs):
  ns = len(bufs)
  for j in range(ns):
    r, gs, _ = bufs[j]

    @pl.when(j < nb)
    def _():
      pltpu.async_copy(table.at[srow(j)], r, gs)

  @pl.loop(0, (maxb + ns - 1) // ns)
  def _(i):
    for j in range(ns):
      r, gs, ss = bufs[j]
      b = i * ns + j

      @pl.when(b < nb)
      def _():
        pltpu.make_async_copy(table.at[srow(b)], r, gs).wait()
        pltpu.async_copy(r, acc.at[drow(b)], ss, add=True)
        nxt = b + ns

        @pl.when(nxt < nb)
        def _():
          pltpu.make_async_copy(r, acc.at[drow(b)], ss).wait()
          pltpu.async_copy(table.at[srow(nxt)], r, gs)

  for j in range(ns):
    r, _, ss = bufs[j]

    @pl.when(nb > j)
    def _():
      lj = nb - 1 - ((nb - 1 - j) % ns)
      pltpu.make_async_copy(r, acc.at[drow(lj)], ss).wait()


def _aggw_body(table, sidx, didx, bcnt, out, sidx_v, didx_v, cnt_v,
               r0, r1, acc, gs0, gs1, ss0, ss1):
  c = lax.axis_index("c")
  s = lax.axis_index("s")
  nfull = TP_TW // BLKW
  rem = TP_TW - nfull * BLKW
  bufs = ((r0, gs0, ss0), (r1, gs1, ss1))

  pltpu.sync_copy(bcnt.at[c].at[s], cnt_v)

  for t in range(TP_P // NC):
    p = t + (TP_P // NC) * c
    lo = p * TP_R

    _fill_f32(r0, BLKW, 128, 0.0)

    @pl.loop(0, nfull)
    def _(m):
      pltpu.sync_copy(r0, acc.at[pl.ds(s * TP_TW + m * BLKW, BLKW)])

    pltpu.sync_copy(r0.at[pl.ds(0, rem)],
                    acc.at[pl.ds(s * TP_TW + nfull * BLKW, rem)])
    plsc.subcore_barrier()

    pltpu.sync_copy(sidx.at[s].at[p], sidx_v)
    pltpu.sync_copy(didx.at[s].at[p], didx_v)
    nb = cnt_v[t, pl.ds(0, 16)][0]
    _edge_loop_dyn(table, acc,
                   lambda b: sidx_v.at[pl.ds(b * BLKW, BLKW)],
                   lambda b: didx_v.at[b],
                   nb, NBK, bufs)

    plsc.subcore_barrier()

    @pl.loop(0, nfull)
    def _(m):
      pltpu.sync_copy(acc.at[pl.ds(s * TP_TW + m * BLKW, BLKW)], r0)
      pltpu.sync_copy(r0, out.at[pl.ds(lo + s * TP_TW + m * BLKW, BLKW)])

    pltpu.sync_copy(acc.at[pl.ds(s * TP_TW + nfull * BLKW, rem)],
                    r0.at[pl.ds(0, rem)])
    pltpu.sync_copy(r0.at[pl.ds(0, rem)],
                    out.at[pl.ds(lo + s * TP_TW + nfull * BLKW, rem)])


_agg_writes = pl.kernel(
    _aggw_body,
    out_type=jax.ShapeDtypeStruct((NTPW, 128), jnp.float32),
    mesh=_mesh,
    scratch_types=[pltpu.VMEM((NBWS,), jnp.int32),
                   pltpu.VMEM((NBK, BLKW), jnp.int32),
                   pltpu.VMEM((8, 16), jnp.int32),
                   pltpu.VMEM((BLKW, 128), jnp.float32),
                   pltpu.VMEM((BLKW, 128), jnp.float32),
                   pltpu.VMEM_SHARED((TP_R + 8, 128), jnp.float32),
                   pltpu.SemaphoreType.DMA,
                   pltpu.SemaphoreType.DMA,
                   pltpu.SemaphoreType.DMA,
                   pltpu.SemaphoreType.DMA],
    name="sc_agg_writes")


BU = 1000
BT = 1000


def _norm_body(sf0, sf1, df0, df1, sw0, sw1, dw0, dw1, osf, odf, osw, odw):
  for a, b, o_ref in ((sf0, sf1, osf), (df0, df1, odf),
                      (sw0, sw1, osw), (dw0, dw1, odw)):
    d = a[...] + b[...]
    o_ref[...] = lax.rsqrt(jnp.maximum(d, 1.0))


_norm_kernel = pl.pallas_call(
    _norm_body,
    out_shape=[jax.ShapeDtypeStruct((1, NUP), jnp.float32),
               jax.ShapeDtypeStruct((1, NUP), jnp.float32),
               jax.ShapeDtypeStruct((1, NUP), jnp.float32),
               jax.ShapeDtypeStruct((1, NTP), jnp.float32)])


def _user_dense_body(ud, un, uc, wd, bd, wn, bn, wc, bc, w10, w11, isf, isw,
                     mu0_o, mu1_o):
  d = _lrelu(jnp.dot(ud[...], wd[...], preferred_element_type=jnp.float32)
             + bd[...])
  n = _lrelu(jnp.dot(un[...], wn[...], preferred_element_type=jnp.float32)
             + bn[...])
  cm = _lrelu(jnp.dot(uc[...], wc[...], preferred_element_type=jnp.float32)
              + bc[...])
  x = jnp.concatenate([d, n, cm], axis=1)
  mu0_o[...] = jnp.dot(x, w10[...], preferred_element_type=jnp.float32) \
      * isf[...]
  mu1_o[...] = jnp.dot(x, w11[...], preferred_element_type=jnp.float32) \
      * isw[...]


_user_dense = pl.pallas_call(
    _user_dense_body,
    grid=(NU // BU,),
    in_specs=[
        pl.BlockSpec((BU, 100), lambda i: (i, 0)),
        pl.BlockSpec((BU, 6), lambda i: (i, 0)),
        pl.BlockSpec((BU, 11), lambda i: (i, 0)),
        pl.BlockSpec((100, 64), lambda i: (0, 0)),
        pl.BlockSpec((1, 64), lambda i: (0, 0)),
        pl.BlockSpec((6, 32), lambda i: (0, 0)),
        pl.BlockSpec((1, 32), lambda i: (0, 0)),
        pl.BlockSpec((11, 32), lambda i: (0, 0)),
        pl.BlockSpec((1, 32), lambda i: (0, 0)),
        pl.BlockSpec((128, 128), lambda i: (0, 0)),
        pl.BlockSpec((128, 128), lambda i: (0, 0)),
        pl.BlockSpec((BU, 1), lambda i: (i, 0)),
        pl.BlockSpec((BU, 1), lambda i: (i, 0)),
    ],
    out_specs=[pl.BlockSpec((BU, 128), lambda i: (i, 0)),
               pl.BlockSpec((BU, 128), lambda i: (i, 0))],
    out_shape=[jax.ShapeDtypeStruct((NU, 128), jnp.float32),
               jax.ShapeDtypeStruct((NU, 128), jnp.float32)])


def _tweet_dense_body(tx, wt, bt_, w12, idw, mt2_o):
  xt = _lrelu(jnp.dot(tx[...], wt[...], preferred_element_type=jnp.float32)
              + bt_[...])
  mt2_o[...] = jnp.dot(xt, w12[...], preferred_element_type=jnp.float32) \
      * idw[...]


_tweet_dense = pl.pallas_call(
    _tweet_dense_body,
    grid=(NT // BT,),
    in_specs=[
        pl.BlockSpec((BT, 100), lambda i: (i, 0)),
        pl.BlockSpec((100, 128), lambda i: (0, 0)),
        pl.BlockSpec((1, 128), lambda i: (0, 0)),
        pl.BlockSpec((128, 128), lambda i: (0, 0)),
        pl.BlockSpec((BT, 1), lambda i: (i, 0)),
    ],
    out_specs=pl.BlockSpec((BT, 128), lambda i: (i, 0)),
    out_shape=jax.ShapeDtypeStruct((NT, 128), jnp.float32))


def _user_mid_body(aggF, aggR, idf, isw, b10, b12, w20, w21, isf,
                   mu0_o, mu1_o):
  u1 = (aggF[0] + aggF[1]) * idf[...] + (aggR[0] + aggR[1]) * isw[...] \
      + b10[...] + b12[...]
  mu0_o[...] = jnp.dot(u1, w20[...], preferred_element_type=jnp.float32) \
      * isf[...]
  mu1_o[...] = jnp.dot(u1, w21[...], preferred_element_type=jnp.float32) \
      * isw[...]


_user_mid = pl.pallas_call(
    _user_mid_body,
    grid=(NU // BU,),
    in_specs=[
        pl.BlockSpec((NC, BU, 128), lambda i: (0, i, 0)),
        pl.BlockSpec((NC, BU, 128), lambda i: (0, i, 0)),
        pl.BlockSpec((BU, 1), lambda i: (i, 0)),
        pl.BlockSpec((BU, 1), lambda i: (i, 0)),
        pl.BlockSpec((1, 128), lambda i: (0, 0)),
        pl.BlockSpec((1, 128), lambda i: (0, 0)),
        pl.BlockSpec((128, 128), lambda i: (0, 0)),
        pl.BlockSpec((128, 128), lambda i: (0, 0)),
        pl.BlockSpec((BU, 1), lambda i: (i, 0)),
    ],
    out_specs=[pl.BlockSpec((BU, 128), lambda i: (i, 0)),
               pl.BlockSpec((BU, 128), lambda i: (i, 0))],
    out_shape=[jax.ShapeDtypeStruct((NU, 128), jnp.float32),
               jax.ShapeDtypeStruct((NU, 128), jnp.float32)])


def _tweet_mid_body(aggW, idw, b11, w22, mt2_o):
  t1 = aggW[...] * idw[...] + b11[...]
  mt2_o[...] = jnp.dot(t1, w22[...], preferred_element_type=jnp.float32) \
      * idw[...]


_tweet_mid = pl.pallas_call(
    _tweet_mid_body,
    grid=(NT // BT,),
    in_specs=[
        pl.BlockSpec((BT, 128), lambda i: (i, 0)),
        pl.BlockSpec((BT, 1), lambda i: (i, 0)),
        pl.BlockSpec((1, 128), lambda i: (0, 0)),
        pl.BlockSpec((128, 128), lambda i: (0, 0)),
    ],
    out_specs=pl.BlockSpec((BT, 128), lambda i: (i, 0)),
    out_shape=jax.ShapeDtypeStruct((NT, 128), jnp.float32))


def _user_head_body(aggF, aggR, idf, isw, b20, b22, wo10, bo10, wo20, bo20,
                    out_o):
  u2 = (aggF[0] + aggF[1]) * idf[...] + (aggR[0] + aggR[1]) * isw[...] \
      + b20[...] + b22[...]
  o = _lrelu(jnp.dot(u2, wo10[...], preferred_element_type=jnp.float32)
             + bo10[...])
  out_o[...] = jnp.dot(o, wo20[...], preferred_element_type=jnp.float32) \
      + bo20[...]


_user_head = pl.pallas_call(
    _user_head_body,
    grid=(NU // BU,),
    in_specs=[
        pl.BlockSpec((NC, BU, 128), lambda i: (0, i, 0)),
        pl.BlockSpec((NC, BU, 128), lambda i: (0, i, 0)),
        pl.BlockSpec((BU, 1), lambda i: (i, 0)),
        pl.BlockSpec((BU, 1), lambda i: (i, 0)),
        pl.BlockSpec((1, 128), lambda i: (0, 0)),
        pl.BlockSpec((1, 128), lambda i: (0, 0)),
        pl.BlockSpec((128, 128), lambda i: (0, 0)),
        pl.BlockSpec((1, 128), lambda i: (0, 0)),
        pl.BlockSpec((128, 2), lambda i: (0, 0)),
        pl.BlockSpec((1, 2), lambda i: (0, 0)),
    ],
    out_specs=pl.BlockSpec((BU, 2), lambda i: (i, 0)),
    out_shape=jax.ShapeDtypeStruct((NU, 2), jnp.float32))


def _tweet_head_body(aggW, idw, b21, wo11, bo11, wo21, bo21, out_o):
  t2 = aggW[...] * idw[...] + b21[...]
  o = _lrelu(jnp.dot(t2, wo11[...], preferred_element_type=jnp.float32)
             + bo11[...])
  out_o[...] = jnp.dot(o, wo21[...], preferred_element_type=jnp.float32) \
      + bo21[...]


_tweet_head = pl.pallas_call(
    _tweet_head_body,
    grid=(NT // BT,),
    in_specs=[
        pl.BlockSpec((BT, 128), lambda i: (i, 0)),
        pl.BlockSpec((BT, 1), lambda i: (i, 0)),
        pl.BlockSpec((1, 128), lambda i: (0, 0)),
        pl.BlockSpec((128, 128), lambda i: (0, 0)),
        pl.BlockSpec((1, 128), lambda i: (0, 0)),
        pl.BlockSpec((128, 2), lambda i: (0, 0)),
        pl.BlockSpec((1, 2), lambda i: (0, 0)),
    ],
    out_specs=pl.BlockSpec((BT, 2), lambda i: (i, 0)),
    out_shape=jax.ShapeDtypeStruct((NT, 2), jnp.float32))


def kernel(user_des, user_num, user_cat, tweet_x, Wd, bd, Wn, bn, Wc, bc,
           Wt, bt, W1, b1, W2, b2, Wo1, bo1, Wo2, bo2,
           edge_index_follows, edge_src_writes, edge_dst_writes):
  sf = edge_index_follows[0]
  df = edge_index_follows[1]
  sw = edge_src_writes
  dw = edge_dst_writes

  def padf(a, total, val):
    p = jnp.full((total - a.shape[0],), val, jnp.int32)
    return jnp.concatenate([a, p])

  sf_pad0 = padf(sf, EFP, 0)
  df_padN = padf(df, EFP, NU)
  sw_pad0 = padf(sw, EWP, 0)
  sw_padN = padf(sw, EWP, NU)
  dw_pad0 = padf(dw, EWP, 0)
  dw_padT = padf(dw, EWP, NT)

  sfN_d = padf(sf, EFP, NU).reshape(NW, EFB, BLK)
  dfN_d = df_padN.reshape(NW, EFB, BLK)
  swN_d = sw_padN.reshape(NW, EWB, BLK)
  dwN_d = dw_padT.reshape(NW, EWB, BLK)
  sf0_a = sf_pad0.reshape(NW, NCHF, CHF, BLKU)
  dfN_a = df_padN.reshape(NW, NCHF, CHF, BLKU)
  dw0_a = dw_pad0.reshape(NW, NCHW, CHW, BLKU)
  swN_a = sw_padN.reshape(NW, NCHW, CHW, BLKU)
  sw0_w = sw_pad0.reshape(NS, NBWS // 128, 128)
  dwT_w = dw_padT.reshape(NS, NBWS // 128, 128)

  dsf0, ddf0, dsw0, ddw0, dsf1, ddf1, dsw1, ddw1 = _deg_kernel(
      sfN_d, dfN_d, swN_d, dwN_d)
  bsrc, bdst, bcnt = _bin_writes(sw0_w, dwT_w)
  r = lambda v: v.reshape(1, -1)
  nsf, ndf, nsw, ndw = _norm_kernel(r(dsf0), r(dsf1), r(ddf0), r(ddf1),
                                    r(dsw0), r(dsw1), r(ddw0), r(ddw1))
  isf = nsf.reshape(NUP, 1)
  idf = ndf.reshape(NUP, 1)
  isw = nsw.reshape(NUP, 1)
  idw = ndw.reshape(NTP, 1)

  mu0, mu1 = _user_dense(user_des, user_num, user_cat, Wd, r(bd), Wn, r(bn),
                         Wc, r(bc), W1[0], W1[1], isf, isw)
  mt2 = _tweet_dense(tweet_x, Wt, r(bt), W1[2], idw)

  aggF1 = _agg_follows(mu0, sf0_a, dfN_a)
  aggR1 = _agg_rev(mt2, dw0_a, swN_a)
  aggW1 = _agg_writes(mu1, bsrc, bdst, bcnt)

  mu0_2, mu1_2 = _user_mid(aggF1, aggR1, idf, isw, r(b1[0]), r(b1[2]),
                           W2[0], W2[1], isf)
  mt2_2 = _tweet_mid(aggW1, idw, r(b1[1]), W2[2])

  aggF2 = _agg_follows(mu0_2, sf0_a, dfN_a)
  aggR2 = _agg_rev(mt2_2, dw0_a, swN_a)
  aggW2 = _agg_writes(mu1_2, bsrc, bdst, bcnt)

  out_u = _user_head(aggF2, aggR2, idf, isw, r(b2[0]), r(b2[2]),
                     Wo1[0], r(bo1[0]), Wo2[0], r(bo2[0]))
  out_t = _tweet_head(aggW2, idw, r(b2[1]), Wo1[1], r(bo1[1]),
                      Wo2[1], r(bo2[1]))

  return jnp.concatenate([out_u, out_t], axis=0)
